# Initial kernel scaffold; baseline (speedup 1.0000x reference)
import jax
import jax.numpy as jnp
from jax.experimental import pallas as pl


def kernel(feature, edge_index, batch, W_head, b_head, W1, b1, g1, be1, p1, W2, b2, g2, be2, p2):
    raise NotImplementedError("write your pallas kernel here")


# speedup vs baseline: 9.4386x; 9.4386x over previous
import functools

import numpy as np
import jax
import jax.numpy as jnp
from jax import lax
from jax.experimental import pallas as pl
from jax.experimental.pallas import tpu as pltpu
from jax.experimental.pallas import tpu_sc as plsc

N = 10000
NP = 10240
E = 320000
H8 = 8
K1 = 2000
K2 = 400
SORT_N = 16384
_BN = np.sqrt(np.float32(1.0) + np.float32(1e-5)).astype(np.float32)
NEG_INF = np.float32(-np.inf)

NC, NS = 2, 16
NW = NC * NS
EP = 327680
EROWS = EP // 128
EPW = EP // NW
ECH = 2048
CROWS = ECH // 128
NCHUNK = EPW // ECH
WROWS = EPW // 128
RPT = NP // NS
NPW = NP * H8
SEG = NPW // NS


def _head_body(f_ref, w_ref, b_ref, e_ref, o_ref, e8_ref):
    fb = f_ref[...].astype(jnp.bfloat16)
    wb = w_ref[...].astype(jnp.bfloat16)
    o = jnp.dot(fb, wb, preferred_element_type=jnp.float32) + b_ref[...]
    o_ref[...] = jnp.maximum(o, 0.0)
    e8_ref[...] = e_ref[...] * H8


def _head(feature, Whp, bhp, edge_index):
    return pl.pallas_call(
        _head_body,
        out_shape=[
            jax.ShapeDtypeStruct((NP, H8), jnp.float32),
            jax.ShapeDtypeStruct((2, E), jnp.int32),
        ],
    )(feature, Whp, bhp, edge_index)


def _segsum_body(x_hbm, s8_hbm, d8_hbm, zer_hbm, out_hbm,
                 i8s_v, i8d_v, ic_v, buf_v, ics_v, vals_v, agg_sp, semg):
    c = lax.axis_index("c")
    s = lax.axis_index("s")
    r0 = s * SEG
    pltpu.sync_copy(zer_hbm.at[pl.ds(r0, SEG)], agg_sp.at[pl.ds(r0, SEG)])
    plsc.subcore_barrier()
    wid = c * NS + s
    for t in range(NCHUNK):
        base = wid * EPW + t * ECH
        pltpu.sync_copy(s8_hbm.at[pl.ds(base, ECH)], i8s_v)
        pltpu.sync_copy(d8_hbm.at[pl.ds(base, ECH)], i8d_v)
        for ch in range(6):

            def gidx(i, _):
                ic_v[pl.ds(i * 16, 16)] = i8s_v[pl.ds(i * 16, 16)] + ch
                return 0

            lax.fori_loop(0, ECH // 16, gidx, 0)
            pltpu.async_copy(x_hbm.at[ic_v], buf_v, semg).wait()

            @pl.loop(0, CROWS)
            def srow(r):
                for l in range(8):
                    ics_v[pl.ds(l * 16, 16)] = (
                        i8d_v[pl.ds(r * 128 + l * 16, 16)] + ch)
                pltpu.sync_copy(buf_v.at[pl.ds(r * 128, 128)],
                                agg_sp.at[ics_v], add=True)
    plsc.subcore_barrier()
    pltpu.sync_copy(agg_sp.at[pl.ds(r0, SEG)], out_hbm.at[c, pl.ds(r0, SEG)])


@functools.cache
def _segsum_kernel():
    return pl.kernel(
        _segsum_body,
        out_type=jax.ShapeDtypeStruct((NC, NPW), jnp.float32),
        mesh=plsc.VectorSubcoreMesh(core_axis_name="c", subcore_axis_name="s"),
        scratch_types=[
            pltpu.VMEM((ECH,), jnp.int32),
            pltpu.VMEM((ECH,), jnp.int32),
            pltpu.VMEM((ECH,), jnp.int32),
            pltpu.VMEM((ECH,), jnp.float32),
            pltpu.VMEM((128,), jnp.int32),
            pltpu.VMEM((128,), jnp.float32),
            pltpu.VMEM_SHARED((NPW,), jnp.float32),
            pltpu.SemaphoreType.DMA,
        ],
    )


def _segsum(x_flat, src8, dst8, zeros_flat):
    return _segsum_kernel()(x_flat, src8, dst8, zeros_flat)


def _dense_body(x_ref, agg_ref, w_ref, b_ref, g_ref, be_ref, p_ref,
                h_ref, s_ref):
    xa = x_ref[...] + agg_ref[0] + agg_ref[1]
    hb = jnp.dot(xa.astype(jnp.bfloat16), w_ref[...].astype(jnp.bfloat16),
                 preferred_element_type=jnp.float32) + b_ref[...]
    hb = hb / _BN * g_ref[...] + be_ref[...]
    h = jnp.maximum(hb, 0.0)
    h_ref[...] = h
    pcol = p_ref[...]
    pnorm = jnp.sqrt(jnp.sum(pcol * pcol))
    sc = jnp.dot(h.astype(jnp.bfloat16), pcol.astype(jnp.bfloat16),
                 preferred_element_type=jnp.float32)
    s_ref[...] = sc / pnorm


def _dense(x, aggp, Wp, bp, gp, bep, pp):
    return pl.pallas_call(
        _dense_body,
        out_shape=[
            jax.ShapeDtypeStruct((NP, H8), jnp.float32),
            jax.ShapeDtypeStruct((NP, 1), jnp.float32),
        ],
    )(x, aggp, Wp, bp, gp, bep, pp)


def _bitonic_desc(key, idx):
    R, C = 128, 128
    row = lax.broadcasted_iota(jnp.int32, (R, C), 0)
    col = lax.broadcasted_iota(jnp.int32, (R, C), 1)
    k = 2
    while k <= R * C:
        j = k // 2
        while j >= 1:
            if j < C:
                pk = jnp.where((col & j) == 0,
                               jnp.roll(key, -j, axis=1), jnp.roll(key, j, axis=1))
                pi = jnp.where((col & j) == 0,
                               jnp.roll(idx, -j, axis=1), jnp.roll(idx, j, axis=1))
                am_low = (col & j) == 0
            else:
                jr = j // C
                pk = jnp.where((row & jr) == 0,
                               jnp.roll(key, -jr, axis=0), jnp.roll(key, jr, axis=0))
                pi = jnp.where((row & jr) == 0,
                               jnp.roll(idx, -jr, axis=0), jnp.roll(idx, jr, axis=0))
                am_low = (row & jr) == 0
            if k < C:
                desc = (col & k) == 0
            else:
                desc = (row & (k // C)) == 0
            beats = (key > pk) | ((key == pk) & (idx < pi))
            take_self = beats == (am_low == desc)
            key = jnp.where(take_self, key, pk)
            idx = jnp.where(take_self, idx, pi)
            j //= 2
        k *= 2
    return key, idx


def _sortx1_body(s2d_ref, h_ref, sc_ref, x1_ref, sel_ref):
    R, C = 128, 128
    row = lax.broadcasted_iota(jnp.int32, (R, C), 0)
    col = lax.broadcasted_iota(jnp.int32, (R, C), 1)
    p = row * C + col
    ks, isrt = _bitonic_desc(s2d_ref[...], p)
    tpos = (row == (K1 - 1) // C) & (col == (K1 - 1) % C)
    ts = jnp.sum(jnp.where(tpos, ks, 0.0))
    ti = jnp.sum(jnp.where(tpos, isrt, 0))
    sc = sc_ref[...]
    nid = lax.broadcasted_iota(jnp.int32, (NP, 1), 0)
    sel = ((sc > ts) | ((sc == ts) & (nid <= ti))) & (nid < N)
    x1_ref[...] = jnp.where(sel, h_ref[...] * jnp.tanh(sc), 0.0)
    sel_ref[...] = sel.astype(jnp.float32)


def _sortx1(score2d, h, score_col):
    return pl.pallas_call(
        _sortx1_body,
        out_shape=[
            jax.ShapeDtypeStruct((NP, H8), jnp.float32),
            jax.ShapeDtypeStruct((NP, 1), jnp.float32),
        ],
    )(score2d, h, score_col)


def _dense2_body(x_ref, agg_ref, w_ref, b_ref, g_ref, be_ref, p_ref, sel_ref,
                 y_ref, k2_ref):
    xa = x_ref[...] + agg_ref[0] + agg_ref[1]
    hb = jnp.dot(xa.astype(jnp.bfloat16), w_ref[...].astype(jnp.bfloat16),
                 preferred_element_type=jnp.float32) + b_ref[...]
    hb = hb / _BN * g_ref[...] + be_ref[...]
    h2 = jnp.maximum(hb, 0.0)
    pcol = p_ref[...]
    pnorm = jnp.sqrt(jnp.sum(pcol * pcol))
    s2 = jnp.dot(h2.astype(jnp.bfloat16), pcol.astype(jnp.bfloat16),
                 preferred_element_type=jnp.float32) / pnorm
    y_ref[...] = h2 * jnp.tanh(s2)
    k2_ref[...] = jnp.where(sel_ref[...] > 0.5, s2, NEG_INF)


def _dense2(x1, agg2p, Wp, bp, gp, bep, pp, selm):
    return pl.pallas_call(
        _dense2_body,
        out_shape=[
            jax.ShapeDtypeStruct((NP, H8), jnp.float32),
            jax.ShapeDtypeStruct((NP, 1), jnp.float32),
        ],
    )(x1, agg2p, Wp, bp, gp, bep, pp, selm)


def _sort2_body(k2d_ref, o_ref):
    R, C = 128, 128
    row = lax.broadcasted_iota(jnp.int32, (R, C), 0)
    col = lax.broadcasted_iota(jnp.int32, (R, C), 1)
    p = row * C + col
    _, isrt = _bitonic_desc(k2d_ref[...], p)
    o_ref[...] = isrt * H8


def _sort2(key2d):
    return pl.pallas_call(
        _sort2_body,
        out_shape=jax.ShapeDtypeStruct((128, 128), jnp.int32),
    )(key2d)


GN = 512
GPW = GN // NW


def _gather_body(y_hbm, idx8_hbm, out_hbm, idx_v, ic_v, buf_v, sem):
    c = lax.axis_index("c")
    s = lax.axis_index("s")
    wid = c * NS + s
    b = wid * GPW
    pltpu.sync_copy(idx8_hbm.at[pl.ds(b, GPW)], idx_v)
    for ch in range(6):
        ic_v[...] = idx_v[...] + ch
        pltpu.async_copy(y_hbm.at[ic_v], buf_v, sem).wait()
        pltpu.sync_copy(buf_v, out_hbm.at[pl.ds(ch * GN + b, GPW)])


@functools.cache
def _gather_kernel():
    return pl.kernel(
        _gather_body,
        out_type=jax.ShapeDtypeStruct((H8 * GN,), jnp.float32),
        mesh=plsc.VectorSubcoreMesh(core_axis_name="c", subcore_axis_name="s"),
        scratch_types=[
            pltpu.VMEM((GPW,), jnp.int32),
            pltpu.VMEM((GPW,), jnp.int32),
            pltpu.VMEM((GPW,), jnp.float32),
            pltpu.SemaphoreType.DMA,
        ],
    )


def _gather(y_flat, idx8):
    return _gather_kernel()(y_flat, idx8)


def kernel(feature, edge_index, batch, W_head, b_head, W1, b1, g1, be1, p1,
           W2, b2, g2, be2, p2):
    f32 = jnp.float32
    src = edge_index[0]
    dst = edge_index[1]
    zeros = jnp.zeros((NP, H8), f32)
    fpad = jnp.concatenate([feature, jnp.zeros((NP - N, 128), f32)], axis=0)

    Whp = jnp.zeros((128, H8), f32).at[:, :6].set(W_head)
    bhp = jnp.zeros((1, H8), f32).at[0, :6].set(b_head)
    W1p = jnp.zeros((H8, H8), f32).at[:6, :6].set(W1)
    b1p = jnp.zeros((1, H8), f32).at[0, :6].set(b1)
    g1p = jnp.zeros((1, H8), f32).at[0, :6].set(g1)
    be1p = jnp.zeros((1, H8), f32).at[0, :6].set(be1)
    p1p = jnp.zeros((H8, 1), f32).at[:6, 0].set(p1)
    W2p = jnp.zeros((H8, H8), f32).at[:6, :6].set(W2)
    b2p = jnp.zeros((1, H8), f32).at[0, :6].set(b2)
    g2p = jnp.zeros((1, H8), f32).at[0, :6].set(g2)
    be2p = jnp.zeros((1, H8), f32).at[0, :6].set(be2)
    p2p = jnp.zeros((H8, 1), f32).at[:6, 0].set(p2)

    x, e8 = _head(fpad, Whp, bhp, edge_index)
    npad = EP - E
    dump8 = (N + (jnp.arange(npad, dtype=jnp.int32) % (NP - N))) * H8
    src8 = jnp.concatenate([e8[0], jnp.zeros((npad,), jnp.int32)])
    dst8 = jnp.concatenate([e8[1], dump8])
    zeros_flat = zeros.reshape(NPW)
    aggp = _segsum(x.reshape(NPW), src8, dst8, zeros_flat)
    h, score_col = _dense(x, aggp.reshape(NC, NP, H8), W1p, b1p, g1p, be1p, p1p)
    pad = jnp.full((SORT_N - N,), NEG_INF, f32)
    score2d = jnp.concatenate([score_col[:N, 0], pad]).reshape(128, 128)
    x1, selm = _sortx1(score2d, h, score_col)
    agg2p = _segsum(x1.reshape(NPW), src8, dst8, zeros_flat)
    y, key2_col = _dense2(x1, agg2p.reshape(NC, NP, H8), W2p, b2p, g2p, be2p, p2p, selm)
    key2d = jnp.concatenate([key2_col[:N, 0], pad]).reshape(128, 128)
    idx2x8 = _sort2(key2d)
    idx512x8 = idx2x8.reshape(SORT_N)[:GN]
    rows_cm = _gather(y.reshape(NPW), idx512x8)
    out_cm = rows_cm.reshape(H8, GN)
    return out_cm.T[:K2, :6]
